# baseline (speedup 1.0000x reference)
import functools

import jax
import jax.numpy as jnp
from jax import lax
from jax.experimental import pallas as pl
from jax.experimental.pallas import tpu as pltpu
from jax.experimental.pallas import tpu_sc as plsc

N = 10000
D = 256
E = 160000
HALF = 128
NSUB = 16
CHUNK = 128
EPW = 10240
KCH = EPW // CHUNK
EPAD = NSUB * EPW

HN = N // 2
HZR = 320
HROWS = NSUB * HZR
QN = HN // 2
QZR = 160
QROWS = NSUB * QZR

NBUF = 4


def _sc_feat_body(table, gsrc, dstc, z128, tok, agg_out,
                  gsrc_v, dst_v, *rest):
    bufs = rest[:NBUF]
    acc = rest[NBUF]
    sems = rest[NBUF + 1:]
    c = lax.axis_index("c")
    s = lax.axis_index("s")

    pltpu.sync_copy(dstc.at[c, s], dst_v)

    for p in (0, 1):
        pltpu.sync_copy(z128, acc.at[pl.ds(s * HZR, HZR)])
        pltpu.sync_copy(gsrc.at[p, s], gsrc_v)
        plsc.subcore_barrier()

        def start(k, b):
            pltpu.async_copy(table.at[gsrc_v.at[k]], bufs[b], sems[b])

        def finish(k, b):
            pltpu.make_async_copy(table.at[gsrc_v.at[k]], bufs[b],
                                  sems[b]).wait()
            pltpu.sync_copy(bufs[b], acc.at[dst_v.at[k]], add=True)

        for b in range(NBUF):
            start(b, b)

        @pl.loop(0, KCH, step=NBUF)
        def _(k):
            for b in range(NBUF):
                finish(k + b, b)

                @pl.when(k + b + NBUF < KCH)
                def _():
                    start(k + b + NBUF, b)

        plsc.subcore_barrier()
        pltpu.sync_copy(acc.at[pl.ds(s * HZR, HZR)],
                        agg_out.at[p, c, pl.ds(s * HZR, HZR)])


def _sc_cnt_body(cdstc, onesh, z128, cnt_out, cdst_v, qdst_v, ones_v, cacc):
    c = lax.axis_index("c")
    s = lax.axis_index("s")
    pltpu.sync_copy(onesh, ones_v)

    for j in (0, 1):
        pltpu.sync_copy(cdstc.at[j, c, s], cdst_v)
        for q in (0, 1):
            pltpu.sync_copy(z128.at[pl.ds(0, QZR)],
                            cacc.at[pl.ds(s * QZR, QZR)])

            @pl.loop(0, KCH)
            def _(k):
                @pl.loop(0, CHUNK, step=16)
                def _(i):
                    t = cdst_v[k, pl.ds(i, 16)] - (QN * q)
                    m = (t >= 0) & (t < QN)
                    qdst_v[k, pl.ds(i, 16)] = jnp.where(m, t, QN)

            plsc.subcore_barrier()

            @pl.loop(0, KCH)
            def _(k):
                pltpu.sync_copy(ones_v, cacc.at[qdst_v.at[k]], add=True)

            plsc.subcore_barrier()
            pltpu.sync_copy(cacc.at[pl.ds(s * QZR, QZR)],
                            cnt_out.at[j, q, c, pl.ds(s * QZR, QZR)])


@functools.cache
def _sc_feat_pass():
    mesh = plsc.VectorSubcoreMesh(core_axis_name="c", subcore_axis_name="s")
    return pl.kernel(
        _sc_feat_body,
        out_type=jax.ShapeDtypeStruct((2, 2, HROWS, HALF), jnp.float32),
        mesh=mesh,
        scratch_types=[
            pltpu.VMEM((KCH, CHUNK), jnp.int32),
            pltpu.VMEM((KCH, CHUNK), jnp.int32),
        ] + [pltpu.VMEM((CHUNK, HALF), jnp.float32)] * NBUF
          + [pltpu.VMEM_SHARED((HROWS, HALF), jnp.float32)]
          + [pltpu.SemaphoreType.DMA] * NBUF)


@functools.cache
def _sc_cnt_pass():
    mesh = plsc.VectorSubcoreMesh(core_axis_name="c", subcore_axis_name="s")
    return pl.kernel(
        _sc_cnt_body,
        out_type=jax.ShapeDtypeStruct((2, 2, 2, QROWS, HALF), jnp.float32),
        mesh=mesh,
        scratch_types=[
            pltpu.VMEM((KCH, CHUNK), jnp.int32),
            pltpu.VMEM((KCH, CHUNK), jnp.int32),
            pltpu.VMEM((CHUNK, HALF), jnp.float32),
            pltpu.VMEM_SHARED((QROWS, HALF), jnp.float32),
        ])


def _gelu(x):
    return 0.5 * x * (1.0 + lax.erf(x * 0.7071067811865476))


def _dot(a, b):
    return lax.dot_general(a, b, (((1,), (0,)), ((), ())),
                           preferred_element_type=jnp.float32,
                           precision=lax.Precision.HIGHEST)


BN = 1000


def _layer_body(classifier, h_ref, agg_ref, cnt_ref, Wsrc_ref,
                bsrc_ref, Wtop_ref, Wbot_ref, bfc_ref, gamma_ref, beta_ref,
                *rest):
    if classifier:
        Wc1_ref, bc1_ref, wc2_ref, bc2_ref, out_ref = rest
    else:
        (out_ref,) = rest
    cnt = cnt_ref[...]
    denom = jnp.maximum(cnt, 1.0)
    mask = jnp.minimum(cnt, 1.0)
    agg = jnp.concatenate([agg_ref[0, 0], agg_ref[1, 0]], axis=-1)
    avg = agg / denom
    neigh = _dot(avg, Wsrc_ref[...]) + mask * bsrc_ref[...]
    pre = (_dot(h_ref[...], Wtop_ref[...]) + _dot(neigh, Wbot_ref[...])
           + bfc_ref[...])
    g = _gelu(pre)
    mu = jnp.mean(g, axis=-1, keepdims=True)
    var = jnp.mean((g - mu) ** 2, axis=-1, keepdims=True)
    hn = (g - mu) * lax.rsqrt(var + 1e-5) * gamma_ref[...] + beta_ref[...]
    if classifier:
        z = _gelu(_dot(hn, Wc1_ref[...]) + bc1_ref[...])
        out_ref[...] = (jnp.sum(z * wc2_ref[...], axis=-1, keepdims=True)
                        + bc2_ref[...])
    else:
        out_ref[...] = hn


def _mk_specs(classifier):
    full = lambda shape: pl.BlockSpec(shape, lambda i: (0,) * len(shape))
    specs = [
        pl.BlockSpec((BN, D), lambda i: (i, 0)),
        pl.BlockSpec((2, 1, BN, HALF),
                     lambda i: (0, i // (HN // BN), i % (HN // BN), 0)),
        pl.BlockSpec((BN, 1), lambda i: (i, 0)),
        full((D, D)), full((1, D)), full((D, D)), full((D, D)),
        full((1, D)), full((1, D)), full((1, D)),
    ]
    if classifier:
        specs += [full((D, D)), full((1, D)), full((1, D)), full((1, 1))]
    return specs


def _layer_tc(h, agg, cnt, Wsrc, bsrc, Wtop, Wbot, bfc, gamma, beta):
    return pl.pallas_call(
        functools.partial(_layer_body, False),
        grid=(N // BN,),
        in_specs=_mk_specs(False),
        out_specs=pl.BlockSpec((BN, D), lambda i: (i, 0)),
        out_shape=jax.ShapeDtypeStruct((N, D), jnp.float32),
    )(h, agg, cnt, Wsrc, bsrc, Wtop, Wbot, bfc, gamma, beta)


def _layer_cls_tc(h, agg, cnt, Wsrc, bsrc, Wtop, Wbot, bfc, gamma, beta,
                  Wc1, bc1, wc2, bc2):
    return pl.pallas_call(
        functools.partial(_layer_body, True),
        grid=(N // BN,),
        in_specs=_mk_specs(True),
        out_specs=pl.BlockSpec((BN, 1), lambda i: (i, 0)),
        out_shape=jax.ShapeDtypeStruct((N, 1), jnp.float32),
    )(h, agg, cnt, Wsrc, bsrc, Wtop, Wbot, bfc, gamma, beta,
      Wc1, bc1, wc2, bc2)


def kernel(x, edge_index1, edge_index2, W_src1, b_src1, W_fc1, b_fc1,
           W_src2, b_src2, W_fc2, b_fc2, gamma, beta, W_c1, b_c1, W_c2, b_c2):
    f32 = jnp.float32
    ei1 = edge_index1.astype(jnp.int32)
    ei2 = edge_index2.astype(jnp.int32)
    pad = EPAD - E

    def prep_gsrc(src):
        b = jnp.concatenate([src * 2, jnp.zeros((pad,), jnp.int32)])
        return jnp.stack([b, b + 1]).reshape(2, NSUB, KCH, CHUNK)

    def prep_dst(dst):
        d = jnp.concatenate([dst, jnp.full((pad,), N, jnp.int32)])
        loc = [jnp.where((d >= HN * c) & (d < HN * (c + 1)), d - HN * c, HN)
               for c in (0, 1)]
        return jnp.stack(loc).reshape(2, NSUB, KCH, CHUNK)

    gsrc1, gsrc2 = prep_gsrc(ei1[0]), prep_gsrc(ei2[0])
    dstc1 = prep_dst(ei1[1])
    dstc2 = prep_dst(ei2[1])
    cdstc = jnp.stack([dstc1, dstc2])
    z128 = jnp.zeros((HZR, HALF), f32)
    onesh = jnp.ones((CHUNK, HALF), f32)

    cnts = _sc_cnt_pass()(cdstc, onesh, z128)

    def cnt_vec(j):
        return jnp.concatenate([cnts[j, 0, 0, :QN, 0], cnts[j, 1, 0, :QN, 0],
                                cnts[j, 0, 1, :QN, 0], cnts[j, 1, 1, :QN, 0]]
                               ).reshape(N, 1)

    agg1 = _sc_feat_pass()(x.reshape(2 * N, HALF), gsrc1, dstc1, z128,
                           x[:8, :16])
    h1 = _layer_tc(x, agg1, cnt_vec(0), W_src1,
                   b_src1.reshape(1, D), W_fc1[:D], W_fc1[D:],
                   b_fc1.reshape(1, D), gamma.reshape(1, D),
                   beta.reshape(1, D))
    agg2 = _sc_feat_pass()(h1.reshape(2 * N, HALF), gsrc2, dstc2, z128,
                           h1[:8, :16])
    out = _layer_cls_tc(h1, agg2, cnt_vec(1), W_src2,
                        b_src2.reshape(1, D), W_fc2[:D], W_fc2[D:],
                        b_fc2.reshape(1, D), gamma.reshape(1, D),
                        beta.reshape(1, D), W_c1, b_c1.reshape(1, D),
                        W_c2.reshape(1, D), b_c2.reshape(1, 1))
    return out


# speedup vs baseline: 1.1964x; 1.1964x over previous
import functools

import jax
import jax.numpy as jnp
from jax import lax
from jax.experimental import pallas as pl
from jax.experimental.pallas import tpu as pltpu
from jax.experimental.pallas import tpu_sc as plsc

N = 10000
D = 256
E = 160000
HALF = 128
NSUB = 16
CHUNK = 128
EPW = 10240
KCH = EPW // CHUNK
EPAD = NSUB * EPW

HN = N // 2
HZR = 320
HROWS = NSUB * HZR

NBUF = 4


def _sc_feat_body(table, gsrc, dstc, z128, tok, agg_out,
                  gsrc_v, dst_v, *rest):
    bufs = rest[:NBUF]
    acc = rest[NBUF]
    sems = rest[NBUF + 1:]
    c = lax.axis_index("c")
    s = lax.axis_index("s")

    pltpu.sync_copy(dstc.at[c, s], dst_v)

    for p in (0, 1):
        pltpu.sync_copy(z128, acc.at[pl.ds(s * HZR, HZR)])
        pltpu.sync_copy(gsrc.at[p, s], gsrc_v)
        plsc.subcore_barrier()

        def start(k, b):
            pltpu.async_copy(table.at[gsrc_v.at[k]], bufs[b], sems[b])

        def finish(k, b):
            pltpu.make_async_copy(table.at[gsrc_v.at[k]], bufs[b],
                                  sems[b]).wait()
            pltpu.sync_copy(bufs[b], acc.at[dst_v.at[k]], add=True)

        for b in range(NBUF):
            start(b, b)

        @pl.loop(0, KCH, step=NBUF)
        def _(k):
            for b in range(NBUF):
                finish(k + b, b)

                @pl.when(k + b + NBUF < KCH)
                def _():
                    start(k + b + NBUF, b)

        plsc.subcore_barrier()
        pltpu.sync_copy(acc.at[pl.ds(s * HZR, HZR)],
                        agg_out.at[p, c, pl.ds(s * HZR, HZR)])


def _sc_cnt_body(cdstc, onesh, z128, cnt_out, cdst_v, ones_v, cacc):
    c = lax.axis_index("c")
    s = lax.axis_index("s")
    pltpu.sync_copy(onesh, ones_v)

    for j in (0, 1):
        pltpu.sync_copy(z128, cacc.at[pl.ds(s * HZR, HZR)])
        pltpu.sync_copy(cdstc.at[j, c, s], cdst_v)
        plsc.subcore_barrier()

        @pl.loop(0, KCH)
        def _(k):
            pltpu.sync_copy(ones_v, cacc.at[cdst_v.at[k]], add=True)

        plsc.subcore_barrier()
        pltpu.sync_copy(cacc.at[pl.ds(s * HZR, HZR)],
                        cnt_out.at[j, c, pl.ds(s * HZR, HZR)])


@functools.cache
def _sc_feat_pass():
    mesh = plsc.VectorSubcoreMesh(core_axis_name="c", subcore_axis_name="s")
    return pl.kernel(
        _sc_feat_body,
        out_type=jax.ShapeDtypeStruct((2, 2, HROWS, HALF), jnp.float32),
        mesh=mesh,
        scratch_types=[
            pltpu.VMEM((KCH, CHUNK), jnp.int32),
            pltpu.VMEM((KCH, CHUNK), jnp.int32),
        ] + [pltpu.VMEM((CHUNK, HALF), jnp.float32)] * NBUF
          + [pltpu.VMEM_SHARED((HROWS, HALF), jnp.float32)]
          + [pltpu.SemaphoreType.DMA] * NBUF)


@functools.cache
def _sc_cnt_pass():
    mesh = plsc.VectorSubcoreMesh(core_axis_name="c", subcore_axis_name="s")
    return pl.kernel(
        _sc_cnt_body,
        out_type=jax.ShapeDtypeStruct((2, 2, HROWS, HALF), jnp.float32),
        mesh=mesh,
        scratch_types=[
            pltpu.VMEM((KCH, CHUNK), jnp.int32),
            pltpu.VMEM((CHUNK, HALF), jnp.float32),
            pltpu.VMEM_SHARED((HROWS, HALF), jnp.float32),
        ])


def _gelu(x):
    return 0.5 * x * (1.0 + lax.erf(x * 0.7071067811865476))


def _dot(a, b):
    return lax.dot_general(a, b, (((1,), (0,)), ((), ())),
                           preferred_element_type=jnp.float32,
                           precision=lax.Precision.HIGHEST)


BN = 1000


def _layer_body(classifier, h_ref, agg_ref, cnt_ref, Wsrc_ref,
                bsrc_ref, Wtop_ref, Wbot_ref, bfc_ref, gamma_ref, beta_ref,
                *rest):
    if classifier:
        Wc1_ref, bc1_ref, wc2_ref, bc2_ref, out_ref = rest
    else:
        (out_ref,) = rest
    cnt = cnt_ref[0, 0, :, 0:1]
    denom = jnp.maximum(cnt, 1.0)
    mask = jnp.minimum(cnt, 1.0)
    agg = jnp.concatenate([agg_ref[0, 0], agg_ref[1, 0]], axis=-1)
    avg = agg / denom
    neigh = _dot(avg, Wsrc_ref[...]) + mask * bsrc_ref[...]
    pre = (_dot(h_ref[...], Wtop_ref[...]) + _dot(neigh, Wbot_ref[...])
           + bfc_ref[...])
    g = _gelu(pre)
    mu = jnp.mean(g, axis=-1, keepdims=True)
    var = jnp.mean((g - mu) ** 2, axis=-1, keepdims=True)
    hn = (g - mu) * lax.rsqrt(var + 1e-5) * gamma_ref[...] + beta_ref[...]
    if classifier:
        z = _gelu(_dot(hn, Wc1_ref[...]) + bc1_ref[...])
        out_ref[...] = (jnp.sum(z * wc2_ref[...], axis=-1, keepdims=True)
                        + bc2_ref[...])
    else:
        out_ref[...] = hn


def _mk_specs(cidx, classifier):
    full = lambda shape: pl.BlockSpec(shape, lambda i: (0,) * len(shape))
    specs = [
        pl.BlockSpec((BN, D), lambda i: (i, 0)),
        pl.BlockSpec((2, 1, BN, HALF),
                     lambda i: (0, i // (HN // BN), i % (HN // BN), 0)),
        pl.BlockSpec((1, 1, BN, HALF),
                     lambda i: (cidx, i // (HN // BN), i % (HN // BN), 0)),
        full((D, D)), full((1, D)), full((D, D)), full((D, D)),
        full((1, D)), full((1, D)), full((1, D)),
    ]
    if classifier:
        specs += [full((D, D)), full((1, D)), full((1, D)), full((1, 1))]
    return specs


def _layer_tc(h, agg, cnt, Wsrc, bsrc, Wtop, Wbot, bfc, gamma, beta):
    return pl.pallas_call(
        functools.partial(_layer_body, False),
        grid=(N // BN,),
        in_specs=_mk_specs(0, False),
        out_specs=pl.BlockSpec((BN, D), lambda i: (i, 0)),
        out_shape=jax.ShapeDtypeStruct((N, D), jnp.float32),
    )(h, agg, cnt, Wsrc, bsrc, Wtop, Wbot, bfc, gamma, beta)


def _layer_cls_tc(h, agg, cnt, Wsrc, bsrc, Wtop, Wbot, bfc, gamma, beta,
                  Wc1, bc1, wc2, bc2):
    return pl.pallas_call(
        functools.partial(_layer_body, True),
        grid=(N // BN,),
        in_specs=_mk_specs(1, True),
        out_specs=pl.BlockSpec((BN, 1), lambda i: (i, 0)),
        out_shape=jax.ShapeDtypeStruct((N, 1), jnp.float32),
    )(h, agg, cnt, Wsrc, bsrc, Wtop, Wbot, bfc, gamma, beta,
      Wc1, bc1, wc2, bc2)


def kernel(x, edge_index1, edge_index2, W_src1, b_src1, W_fc1, b_fc1,
           W_src2, b_src2, W_fc2, b_fc2, gamma, beta, W_c1, b_c1, W_c2, b_c2):
    f32 = jnp.float32
    ei1 = edge_index1.astype(jnp.int32)
    ei2 = edge_index2.astype(jnp.int32)
    pad = EPAD - E

    def prep_gsrc(src):
        b = jnp.concatenate([src * 2, jnp.zeros((pad,), jnp.int32)])
        return jnp.stack([b, b + 1]).reshape(2, NSUB, KCH, CHUNK)

    def prep_dst(dst):
        d = jnp.concatenate([dst, jnp.full((pad,), N, jnp.int32)])
        loc = [jnp.where((d >= HN * c) & (d < HN * (c + 1)), d - HN * c, HN)
               for c in (0, 1)]
        return jnp.stack(loc).reshape(2, NSUB, KCH, CHUNK)

    gsrc1, gsrc2 = prep_gsrc(ei1[0]), prep_gsrc(ei2[0])
    dstc1 = prep_dst(ei1[1])
    dstc2 = prep_dst(ei2[1])
    cdstc = jnp.stack([dstc1, dstc2])
    z128 = jnp.zeros((HZR, HALF), f32)
    onesh = jnp.ones((CHUNK, HALF), f32)

    cnts = _sc_cnt_pass()(cdstc, onesh, z128)
    agg1 = _sc_feat_pass()(x.reshape(2 * N, HALF), gsrc1, dstc1, z128,
                           cnts[0, 0, :8, :16])
    h1 = _layer_tc(x, agg1, cnts, W_src1,
                   b_src1.reshape(1, D), W_fc1[:D], W_fc1[D:],
                   b_fc1.reshape(1, D), gamma.reshape(1, D),
                   beta.reshape(1, D))
    agg2 = _sc_feat_pass()(h1.reshape(2 * N, HALF), gsrc2, dstc2, z128,
                           cnts[1, 0, :8, :16])
    out = _layer_cls_tc(h1, agg2, cnts, W_src2,
                        b_src2.reshape(1, D), W_fc2[:D], W_fc2[D:],
                        b_fc2.reshape(1, D), gamma.reshape(1, D),
                        beta.reshape(1, D), W_c1, b_c1.reshape(1, D),
                        W_c2.reshape(1, D), b_c2.reshape(1, 1))
    return out
